# Initial kernel scaffold; baseline (speedup 1.0000x reference)
import jax
import jax.numpy as jnp
from jax.experimental import pallas as pl


def kernel(x, edge_index, edge_attr, W_in, WQ, WK, WV, WE, W_out, b_out, gn_weight, gn_bias, gn_alpha, W_cls, b_cls):
    raise NotImplementedError("write your pallas kernel here")


# speedup vs baseline: 16.9005x; 16.9005x over previous
import functools
import jax
import jax.numpy as jnp
from jax import lax
from jax.experimental import pallas as pl
from jax.experimental.pallas import tpu as pltpu
from jax.experimental.pallas import tpu_sc as plsc

N = 10000
E = 320000
HID = 128
HEADS = 8
HD = 16
EIN = 16
HALF = 64

ROWS = 400
GRID = N // ROWS
EROWS = 1600
EGRID = E // EROWS

NS = 16
EPT = E // NS
B = 80
NCHUNK = EPT // B
EG = 16

UR = 5000
UPT = 320
UPT_LAST = UR - UPT * (NS - 1)
RSR = 320
RPT = 32

f32 = jnp.float32

_GDN = lax.GatherDimensionNumbers(offset_dims=(), collapsed_slice_dims=(0,),
                                  start_index_map=(0,))


def _perm(x, idx):
    return lax.gather(x, idx[:, None], _GDN, (1,),
                      mode=lax.GatherScatterMode.PROMISE_IN_BOUNDS)


def _hsum(x, lane):
    for sh in (1, 2, 4, 8):
        x = x + _perm(x, lane ^ sh)
    return x


def _prework_body(x_ref, win_ref, wq_ref, wk_ref, wv_ref,
                  h_ref, q_ref, kv0_ref, kv1_ref):
    h = jnp.dot(x_ref[...], win_ref[...], preferred_element_type=f32)
    h_ref[...] = h
    q_ref[...] = jnp.dot(h, wq_ref[...], preferred_element_type=f32)
    k = jnp.dot(h, wk_ref[...], preferred_element_type=f32)
    v = jnp.dot(h, wv_ref[...], preferred_element_type=f32)
    kv0_ref[...] = jnp.concatenate([k[:, :HALF], v[:, :HALF]], axis=1)
    kv1_ref[...] = jnp.concatenate([k[:, HALF:], v[:, HALF:]], axis=1)


def _prework(x, W_in, WQ, WK, WV):
    w_spec = pl.BlockSpec((HID, HID), lambda i: (0, 0))
    r128 = pl.BlockSpec((ROWS, HID), lambda i: (i, 0))
    return pl.pallas_call(
        _prework_body,
        grid=(GRID,),
        in_specs=[r128, w_spec, w_spec, w_spec, w_spec],
        out_specs=[r128, r128, r128, r128],
        out_shape=[
            jax.ShapeDtypeStruct((N, HID), f32),
            jax.ShapeDtypeStruct((N, HID), f32),
            jax.ShapeDtypeStruct((N, HID), f32),
            jax.ShapeDtypeStruct((N, HID), f32),
        ],
    )(x, W_in, WQ, WK, WV)


def _emb_body(ea_ref, we_ref, e0_ref, e1_ref):
    emb = jnp.dot(ea_ref[...], we_ref[...], preferred_element_type=f32)
    e0_ref[...] = emb[:, :HALF]
    e1_ref[...] = emb[:, HALF:]


def _prework_edges(edge_attr, WE):
    return pl.pallas_call(
        _emb_body,
        grid=(EGRID,),
        in_specs=[pl.BlockSpec((EROWS, EIN), lambda i: (i, 0)),
                  pl.BlockSpec((EIN, HID), lambda i: (0, 0))],
        out_specs=[pl.BlockSpec((EROWS, HALF), lambda i: (i, 0)),
                   pl.BlockSpec((EROWS, HALF), lambda i: (i, 0))],
        out_shape=[jax.ShapeDtypeStruct((E, HALF), f32),
                   jax.ShapeDtypeStruct((E, HALF), f32)],
    )(edge_attr, WE)


def _edge_body(zeros_ref, src_ref, dst_ref, q_ref, emb0, emb1, kv0, kv1,
               u0, u1, rs0, rs1,
               idx_src, idx_dst, idx2, idx8, q_buf, kv_buf, emb_buf,
               mt_buf, rs2_buf,
               u_s, rs_s):
    cid = lax.axis_index("c")
    sid = lax.axis_index("s")
    lane = lax.iota(jnp.int32, 16)
    zero16 = jnp.zeros((16,), f32)

    ubase = sid * UPT
    rbase = sid * RPT

    @pl.when(sid < NS - 1)
    def _():
        pltpu.sync_copy(zeros_ref.at[pl.ds(0, UPT)], u_s.at[pl.ds(ubase, UPT)])

    @pl.when(sid == NS - 1)
    def _():
        pltpu.sync_copy(zeros_ref.at[pl.ds(0, UPT_LAST)],
                        u_s.at[pl.ds(ubase, UPT_LAST)])

    @pl.when(sid < 10)
    def _():
        pltpu.sync_copy(zeros_ref.at[pl.ds(0, RPT)], rs_s.at[pl.ds(rbase, RPT)])

    plsc.subcore_barrier()

    ebase = sid * EPT

    def _edge_pass(emb_t, kv_t, qoff):
        def chunk(ci, c):
            eb = ebase + ci * B
            pltpu.sync_copy(src_ref.at[pl.ds(eb, B)], idx_src)
            pltpu.sync_copy(dst_ref.at[pl.ds(eb, B)], idx_dst)
            pltpu.sync_copy(emb_t.at[pl.ds(eb, B)], emb_buf)
            pltpu.sync_copy(q_ref.at[idx_dst], q_buf)
            pltpu.sync_copy(kv_t.at[idx_src], kv_buf)

            def rowidx(i, cc):
                d16 = idx_dst[pl.ds(i * 16, 16)]
                idx2[pl.ds(i * 16, 16)] = lax.shift_right_logical(d16, 1)
                idx8[pl.ds(i * 16, 16)] = lax.shift_right_logical(d16, 5)
                return cc

            lax.fori_loop(0, B // 16, rowidx, 0)

            def group(g, cc):
                dst16 = idx_dst[pl.ds(g * EG, EG)]
                for j in range(EG):
                    e = g * EG + j
                    dvec = _perm(dst16, jnp.full((16,), j, jnp.int32))
                    sf1 = jnp.bitwise_and(dvec, 1).astype(f32)
                    svec = zero16
                    for hh in range(4):
                        qh = q_buf[e, pl.ds(qoff + hh * 16, 16)]
                        kh = kv_buf[e, pl.ds(hh * 16, 16)]
                        vh = kv_buf[e, pl.ds(HALF + hh * 16, 16)]
                        eh = emb_buf[e, pl.ds(hh * 16, 16)]
                        tot = _hsum(qh * (kh + eh), lane)
                        sb = jnp.maximum(tot * 0.25, 0.0)
                        svec = jnp.where(lane == hh, sb, svec)
                        msg = sb * (vh + eh)
                        mt_buf[e, pl.ds(hh * 16, 16)] = msg * (1.0 - sf1)
                        mt_buf[e, pl.ds(HALF + hh * 16, 16)] = msg * sf1
                    tvec = jnp.bitwise_and(dvec, 3)
                    rot = _perm(svec, jnp.bitwise_and(lane - 4 * tvec, 15))
                    segf = jnp.bitwise_and(lax.shift_right_logical(dvec, 2),
                                           7).astype(f32)
                    for ss in range(8):
                        d = segf - float(ss)
                        ind = jnp.maximum(1.0 - d * d, 0.0)
                        rs2_buf[e, pl.ds(ss * 16, 16)] = rot * ind
                return cc

            lax.fori_loop(0, B // EG, group, 0)
            pltpu.sync_copy(mt_buf, u_s.at[idx2], add=True)
            pltpu.sync_copy(rs2_buf, rs_s.at[idx8], add=True)
            return c

        lax.fori_loop(0, NCHUNK, chunk, 0)

    def _copy_out(u_out, rs_out):
        plsc.subcore_barrier()

        @pl.when(sid < NS - 1)
        def _():
            pltpu.sync_copy(u_s.at[pl.ds(ubase, UPT)],
                            u_out.at[pl.ds(ubase, UPT)])

        @pl.when(sid == NS - 1)
        def _():
            pltpu.sync_copy(u_s.at[pl.ds(ubase, UPT_LAST)],
                            u_out.at[pl.ds(ubase, UPT_LAST)])

        @pl.when(sid < 10)
        def _():
            pltpu.sync_copy(rs_s.at[pl.ds(rbase, RPT)],
                            rs_out.at[pl.ds(rbase, RPT)])

    @pl.when(cid == 0)
    def _():
        _edge_pass(emb0, kv0, 0)
        _copy_out(u0, rs0)

    @pl.when(cid == 1)
    def _():
        _edge_pass(emb1, kv1, HALF)
        _copy_out(u1, rs1)


def _edge_phase(zeros, src, dst, q, emb0, emb1, kv0, kv1):
    fn = pl.kernel(
        _edge_body,
        out_type=[
            jax.ShapeDtypeStruct((UR, HID), f32),
            jax.ShapeDtypeStruct((UR, HID), f32),
            jax.ShapeDtypeStruct((RSR, HID), f32),
            jax.ShapeDtypeStruct((RSR, HID), f32),
        ],
        mesh=plsc.VectorSubcoreMesh(core_axis_name="c", subcore_axis_name="s"),
        scratch_types=[
            pltpu.VMEM((B,), jnp.int32),
            pltpu.VMEM((B,), jnp.int32),
            pltpu.VMEM((B,), jnp.int32),
            pltpu.VMEM((B,), jnp.int32),
            pltpu.VMEM((B, HID), f32),
            pltpu.VMEM((B, HID), f32),
            pltpu.VMEM((B, HALF), f32),
            pltpu.VMEM((B, HID), f32),
            pltpu.VMEM((B, HID), f32),
            pltpu.VMEM_SHARED((UR, HID), f32),
            pltpu.VMEM_SHARED((RSR, HID), f32),
        ],
    )
    return fn(zeros, src, dst, q, emb0, emb1, kv0, kv1)


def _epi1_body(u0, u1, rs_ref, h_ref, wo_ref, exp_ref, bo_ref,
               out_ref, cs_ref, cq_ref):
    i = pl.program_id(0)
    u = jnp.concatenate([u0[...], u1[...]], axis=1)
    denom = jnp.dot(rs_ref[...], exp_ref[...], preferred_element_type=f32) + 1e-6
    aggf = u / denom
    out = jnp.dot(aggf, wo_ref[...], preferred_element_type=f32)
    out = out + bo_ref[...] + h_ref[...]
    out_ref[...] = out

    @pl.when(i == 0)
    def _():
        cs_ref[...] = jnp.zeros_like(cs_ref)
        cq_ref[...] = jnp.zeros_like(cq_ref)

    cs_ref[...] += jnp.sum(out, axis=0, keepdims=True)
    cq_ref[...] += jnp.sum(out * out, axis=0, keepdims=True)


def _epilogue1(u0h, u1h, rsfull, h, W_out, EXP, b_out2):
    r128 = pl.BlockSpec((ROWS, HID), lambda i: (i, 0))
    r64 = pl.BlockSpec((ROWS, HALF), lambda i: (i, 0))
    r8 = pl.BlockSpec((ROWS, 8), lambda i: (i, 0))
    w_spec = pl.BlockSpec((HID, HID), lambda i: (0, 0))
    e_spec = pl.BlockSpec((8, HID), lambda i: (0, 0))
    one_spec = pl.BlockSpec((1, HID), lambda i: (0, 0))
    return pl.pallas_call(
        _epi1_body,
        grid=(GRID,),
        in_specs=[r64, r64, r8, r128, w_spec, e_spec, one_spec],
        out_specs=[r128, one_spec, one_spec],
        out_shape=[
            jax.ShapeDtypeStruct((N, HID), f32),
            jax.ShapeDtypeStruct((1, HID), f32),
            jax.ShapeDtypeStruct((1, HID), f32),
        ],
    )(u0h, u1h, rsfull, h, W_out, EXP, b_out2)


def _epi2_body(out_ref, cs_ref, cq_ref, gw_ref, gb_ref, ga_ref, wc_ref, bc_ref,
               lg_ref):
    inv_n = 1.0 / N
    mean = cs_ref[...] * inv_n
    eo2 = cq_ref[...] * inv_n
    a = ga_ref[...]
    var = eo2 - (2.0 * a - a * a) * mean * mean
    inv = lax.rsqrt(var + 1e-5)
    o = out_ref[...]
    sh = o - a * mean
    nrm = gw_ref[...] * sh * inv + gb_ref[...]
    nrm = jnp.maximum(nrm, 0.0)
    lg_ref[...] = jnp.dot(nrm, wc_ref[...], preferred_element_type=f32) + bc_ref[...]


def _epilogue2(out, cs, cq, gw2, gb2, ga2, W_cls_p, b_cls_p):
    r128 = pl.BlockSpec((ROWS, HID), lambda i: (i, 0))
    w_spec = pl.BlockSpec((HID, HID), lambda i: (0, 0))
    one_spec = pl.BlockSpec((1, HID), lambda i: (0, 0))
    return pl.pallas_call(
        _epi2_body,
        grid=(GRID,),
        in_specs=[r128, one_spec, one_spec, one_spec, one_spec, one_spec,
                  w_spec, one_spec],
        out_specs=r128,
        out_shape=jax.ShapeDtypeStruct((N, HID), f32),
    )(out, cs, cq, gw2, gb2, ga2, W_cls_p, b_cls_p)


@jax.jit
def kernel(x, edge_index, edge_attr, W_in, WQ, WK, WV, WE, W_out, b_out,
           gn_weight, gn_bias, gn_alpha, W_cls, b_cls):
    EXP = jnp.kron(jnp.eye(HEADS, dtype=f32), jnp.ones((1, HD), f32))

    h, q, kv0, kv1 = _prework(x, W_in, WQ, WK, WV)
    emb0, emb1 = _prework_edges(edge_attr, WE)
    zeros = jnp.zeros((UPT, HID), f32)
    u0, u1, rs0, rs1 = _edge_phase(zeros, edge_index[0], edge_index[1], q,
                                   emb0, emb1, kv0, kv1)
    u0h = u0.reshape(-1, HALF)[:N]
    u1h = u1.reshape(-1, HALF)[:N]
    rs0u = rs0.reshape(-1, 4)[:N]
    rs1u = rs1.reshape(-1, 4)[:N]
    rsfull = jnp.concatenate([rs0u, rs1u], axis=1)
    out, cs, cq = _epilogue1(u0h, u1h, rsfull, h, W_out, EXP,
                             b_out.reshape(1, HID))
    W_cls_p = jnp.zeros((HID, HID), f32).at[:, :2].set(W_cls)
    b_cls_p = jnp.zeros((1, HID), f32).at[0, :2].set(b_cls)
    logits_p = _epilogue2(out, cs, cq, gn_weight.reshape(1, HID),
                          gn_bias.reshape(1, HID), gn_alpha.reshape(1, HID),
                          W_cls_p, b_cls_p)
    return logits_p[:, :2]


# speedup vs baseline: 17.9352x; 1.0612x over previous
import functools
import jax
import jax.numpy as jnp
from jax import lax
from jax.experimental import pallas as pl
from jax.experimental.pallas import tpu as pltpu
from jax.experimental.pallas import tpu_sc as plsc

N = 10000
E = 320000
HID = 128
HEADS = 8
HD = 16
EIN = 16
HALF = 64

ROWS = 400
GRID = N // ROWS
EROWS = 1600
EGRID = E // EROWS

NS = 16
EPT = E // NS
B = 80
NCHUNK = EPT // B
EG = 16

UR = 5000
UPT = 320
UPT_LAST = UR - UPT * (NS - 1)
RSR = 320
RPT = 32

f32 = jnp.float32

_GDN = lax.GatherDimensionNumbers(offset_dims=(), collapsed_slice_dims=(0,),
                                  start_index_map=(0,))


def _perm(x, idx):
    return lax.gather(x, idx[:, None], _GDN, (1,),
                      mode=lax.GatherScatterMode.PROMISE_IN_BOUNDS)


def _hsum(x, lane):
    for sh in (1, 2, 4, 8):
        x = x + _perm(x, lane ^ sh)
    return x


def _prework_body(x_ref, win_ref, wq_ref, wk_ref, wv_ref,
                  h_ref, q_ref, kv0_ref, kv1_ref):
    h = jnp.dot(x_ref[...], win_ref[...], preferred_element_type=f32)
    h_ref[...] = h
    q_ref[...] = jnp.dot(h, wq_ref[...], preferred_element_type=f32)
    k = jnp.dot(h, wk_ref[...], preferred_element_type=f32)
    v = jnp.dot(h, wv_ref[...], preferred_element_type=f32)
    kv0_ref[...] = jnp.concatenate([k[:, :HALF], v[:, :HALF]], axis=1)
    kv1_ref[...] = jnp.concatenate([k[:, HALF:], v[:, HALF:]], axis=1)


def _prework(x, W_in, WQ, WK, WV):
    w_spec = pl.BlockSpec((HID, HID), lambda i: (0, 0))
    r128 = pl.BlockSpec((ROWS, HID), lambda i: (i, 0))
    return pl.pallas_call(
        _prework_body,
        grid=(GRID,),
        in_specs=[r128, w_spec, w_spec, w_spec, w_spec],
        out_specs=[r128, r128, r128, r128],
        out_shape=[
            jax.ShapeDtypeStruct((N, HID), f32),
            jax.ShapeDtypeStruct((N, HID), f32),
            jax.ShapeDtypeStruct((N, HID), f32),
            jax.ShapeDtypeStruct((N, HID), f32),
        ],
    )(x, W_in, WQ, WK, WV)


def _emb_body(ea_ref, we_ref, e0_ref, e1_ref):
    emb = jnp.dot(ea_ref[...], we_ref[...], preferred_element_type=f32)
    e0_ref[...] = emb[:, :HALF]
    e1_ref[...] = emb[:, HALF:]


def _prework_edges(edge_attr, WE):
    return pl.pallas_call(
        _emb_body,
        grid=(EGRID,),
        in_specs=[pl.BlockSpec((EROWS, EIN), lambda i: (i, 0)),
                  pl.BlockSpec((EIN, HID), lambda i: (0, 0))],
        out_specs=[pl.BlockSpec((EROWS, HALF), lambda i: (i, 0)),
                   pl.BlockSpec((EROWS, HALF), lambda i: (i, 0))],
        out_shape=[jax.ShapeDtypeStruct((E, HALF), f32),
                   jax.ShapeDtypeStruct((E, HALF), f32)],
    )(edge_attr, WE)


def _edge_body(zeros_ref, src_ref, dst_ref, q_ref, emb0, emb1, kv0, kv1,
               u0, u1, rs0, rs1,
               idx_src0, idx_dst0, q_buf0, kv_buf0, emb_buf0,
               idx_src1, idx_dst1, q_buf1, kv_buf1, emb_buf1,
               idx2, idx8, mt_buf, rs2_buf,
               sA0, sB0, sE0, sA1, sB1, sE1, sC,
               u_s, rs_s):
    cid = lax.axis_index("c")
    sid = lax.axis_index("s")
    lane = lax.iota(jnp.int32, 16)
    zero16 = jnp.zeros((16,), f32)

    ubase = sid * UPT
    rbase = sid * RPT

    @pl.when(sid < NS - 1)
    def _():
        pltpu.sync_copy(zeros_ref.at[pl.ds(0, UPT)], u_s.at[pl.ds(ubase, UPT)])

    @pl.when(sid == NS - 1)
    def _():
        pltpu.sync_copy(zeros_ref.at[pl.ds(0, UPT_LAST)],
                        u_s.at[pl.ds(ubase, UPT_LAST)])

    @pl.when(sid < 10)
    def _():
        pltpu.sync_copy(zeros_ref.at[pl.ds(0, RPT)], rs_s.at[pl.ds(rbase, RPT)])

    plsc.subcore_barrier()

    ebase = sid * EPT

    BUFS = ((idx_src0, idx_dst0, q_buf0, kv_buf0, emb_buf0, sA0, sB0, sE0),
            (idx_src1, idx_dst1, q_buf1, kv_buf1, emb_buf1, sA1, sB1, sE1))

    def _edge_pass(emb_t, kv_t, qoff):
        def start_in(s, ci):
            isrc, idst, _, _, embb, sa, _, se = BUFS[s]
            eb = ebase + ci * B
            pltpu.async_copy(src_ref.at[pl.ds(eb, B)], isrc, sa)
            pltpu.async_copy(dst_ref.at[pl.ds(eb, B)], idst, sa)
            pltpu.async_copy(emb_t.at[pl.ds(eb, B)], embb, se)

        def wait_in(s):
            isrc, idst, _, _, _, sa, _, _ = BUFS[s]
            pltpu.make_async_copy(src_ref.at[pl.ds(0, B)], isrc, sa).wait()
            pltpu.make_async_copy(dst_ref.at[pl.ds(0, B)], idst, sa).wait()

        def start_gather(s):
            isrc, idst, qb, kvb, _, _, sb_, _ = BUFS[s]
            pltpu.async_copy(q_ref.at[idst], qb, sb_)
            pltpu.async_copy(kv_t.at[isrc], kvb, sb_)

        def wait_gather(s):
            isrc, idst, qb, kvb, embb, _, sb_, se = BUFS[s]
            pltpu.make_async_copy(q_ref.at[idst], qb, sb_).wait()
            pltpu.make_async_copy(kv_t.at[isrc], kvb, sb_).wait()
            pltpu.make_async_copy(emb_t.at[pl.ds(0, B)], embb, se).wait()

        def compute(s):
            _, idst, qb, kvb, embb, _, _, _ = BUFS[s]

            def rowidx(i, cc):
                d16 = idst[pl.ds(i * 16, 16)]
                idx2[pl.ds(i * 16, 16)] = lax.shift_right_logical(d16, 1)
                idx8[pl.ds(i * 16, 16)] = lax.shift_right_logical(d16, 5)
                return cc

            lax.fori_loop(0, B // 16, rowidx, 0)

            def group(g, cc):
                dst16 = idst[pl.ds(g * EG, EG)]

                def edge(j, c2):
                    e = g * EG + j
                    dvec = _perm(dst16, jnp.full((16,), j, jnp.int32))
                    sf1 = jnp.bitwise_and(dvec, 1).astype(f32)
                    svec = zero16
                    for hh in range(4):
                        qh = qb[e, pl.ds(qoff + hh * 16, 16)]
                        kh = kvb[e, pl.ds(hh * 16, 16)]
                        vh = kvb[e, pl.ds(HALF + hh * 16, 16)]
                        eh = embb[e, pl.ds(hh * 16, 16)]
                        tot = _hsum(qh * (kh + eh), lane)
                        sb = jnp.maximum(tot * 0.25, 0.0)
                        svec = jnp.where(lane == hh, sb, svec)
                        msg = sb * (vh + eh)
                        mt_buf[e, pl.ds(hh * 16, 16)] = msg * (1.0 - sf1)
                        mt_buf[e, pl.ds(HALF + hh * 16, 16)] = msg * sf1
                    tvec = jnp.bitwise_and(dvec, 3)
                    rot = _perm(svec, jnp.bitwise_and(lane - 4 * tvec, 15))
                    segf = jnp.bitwise_and(lax.shift_right_logical(dvec, 2),
                                           7).astype(f32)
                    for ss in range(8):
                        d = segf - float(ss)
                        ind = jnp.maximum(1.0 - d * d, 0.0)
                        rs2_buf[e, pl.ds(ss * 16, 16)] = rot * ind
                    return c2

                lax.fori_loop(0, EG, edge, 0)
                return cc

            lax.fori_loop(0, B // EG, group, 0)

        def wait_scatter():
            pltpu.make_async_copy(mt_buf, u_s.at[idx2], sC).wait()
            pltpu.make_async_copy(rs2_buf, rs_s.at[idx8], sC).wait()

        def start_scatter():
            pltpu.async_copy(mt_buf, u_s.at[idx2], sC, add=True)
            pltpu.async_copy(rs2_buf, rs_s.at[idx8], sC, add=True)

        start_in(0, 0)
        wait_in(0)
        start_gather(0)
        start_in(1, 1)

        def pair(pi, c):
            for b_ in (0, 1):
                i = 2 * pi + b_
                s, o = b_, 1 - b_

                @pl.when(i + 1 < NCHUNK)
                def _():
                    wait_in(o)
                    start_gather(o)

                wait_gather(s)

                @pl.when(i > 0)
                def _():
                    wait_scatter()

                compute(s)
                start_scatter()

                @pl.when(i + 2 < NCHUNK)
                def _():
                    start_in(s, i + 2)
            return c

        lax.fori_loop(0, NCHUNK // 2, pair, 0)
        wait_scatter()

    def _copy_out(u_out, rs_out):
        plsc.subcore_barrier()

        @pl.when(sid < NS - 1)
        def _():
            pltpu.sync_copy(u_s.at[pl.ds(ubase, UPT)],
                            u_out.at[pl.ds(ubase, UPT)])

        @pl.when(sid == NS - 1)
        def _():
            pltpu.sync_copy(u_s.at[pl.ds(ubase, UPT_LAST)],
                            u_out.at[pl.ds(ubase, UPT_LAST)])

        @pl.when(sid < 10)
        def _():
            pltpu.sync_copy(rs_s.at[pl.ds(rbase, RPT)],
                            rs_out.at[pl.ds(rbase, RPT)])

    @pl.when(cid == 0)
    def _():
        _edge_pass(emb0, kv0, 0)
        _copy_out(u0, rs0)

    @pl.when(cid == 1)
    def _():
        _edge_pass(emb1, kv1, HALF)
        _copy_out(u1, rs1)


def _edge_phase(zeros, src, dst, q, emb0, emb1, kv0, kv1):
    fn = pl.kernel(
        _edge_body,
        out_type=[
            jax.ShapeDtypeStruct((UR, HID), f32),
            jax.ShapeDtypeStruct((UR, HID), f32),
            jax.ShapeDtypeStruct((RSR, HID), f32),
            jax.ShapeDtypeStruct((RSR, HID), f32),
        ],
        mesh=plsc.VectorSubcoreMesh(core_axis_name="c", subcore_axis_name="s"),
        scratch_types=(
            [
                pltpu.VMEM((B,), jnp.int32),
                pltpu.VMEM((B,), jnp.int32),
                pltpu.VMEM((B, HID), f32),
                pltpu.VMEM((B, HID), f32),
                pltpu.VMEM((B, HALF), f32),
            ] * 2
            + [
                pltpu.VMEM((B,), jnp.int32),
                pltpu.VMEM((B,), jnp.int32),
                pltpu.VMEM((B, HID), f32),
                pltpu.VMEM((B, HID), f32),
            ]
            + [pltpu.SemaphoreType.DMA] * 7
            + [
                pltpu.VMEM_SHARED((UR, HID), f32),
                pltpu.VMEM_SHARED((RSR, HID), f32),
            ]
        ),
    )
    return fn(zeros, src, dst, q, emb0, emb1, kv0, kv1)


def _epi1_body(u0, u1, rs_ref, h_ref, wo_ref, exp_ref, bo_ref,
               out_ref, cs_ref, cq_ref):
    i = pl.program_id(0)
    u = jnp.concatenate([u0[...], u1[...]], axis=1)
    denom = jnp.dot(rs_ref[...], exp_ref[...], preferred_element_type=f32) + 1e-6
    aggf = u / denom
    out = jnp.dot(aggf, wo_ref[...], preferred_element_type=f32)
    out = out + bo_ref[...] + h_ref[...]
    out_ref[...] = out

    @pl.when(i == 0)
    def _():
        cs_ref[...] = jnp.zeros_like(cs_ref)
        cq_ref[...] = jnp.zeros_like(cq_ref)

    cs_ref[...] += jnp.sum(out, axis=0, keepdims=True)
    cq_ref[...] += jnp.sum(out * out, axis=0, keepdims=True)


def _epilogue1(u0h, u1h, rsfull, h, W_out, EXP, b_out2):
    r128 = pl.BlockSpec((ROWS, HID), lambda i: (i, 0))
    r64 = pl.BlockSpec((ROWS, HALF), lambda i: (i, 0))
    r8 = pl.BlockSpec((ROWS, 8), lambda i: (i, 0))
    w_spec = pl.BlockSpec((HID, HID), lambda i: (0, 0))
    e_spec = pl.BlockSpec((8, HID), lambda i: (0, 0))
    one_spec = pl.BlockSpec((1, HID), lambda i: (0, 0))
    return pl.pallas_call(
        _epi1_body,
        grid=(GRID,),
        in_specs=[r64, r64, r8, r128, w_spec, e_spec, one_spec],
        out_specs=[r128, one_spec, one_spec],
        out_shape=[
            jax.ShapeDtypeStruct((N, HID), f32),
            jax.ShapeDtypeStruct((1, HID), f32),
            jax.ShapeDtypeStruct((1, HID), f32),
        ],
    )(u0h, u1h, rsfull, h, W_out, EXP, b_out2)


def _epi2_body(out_ref, cs_ref, cq_ref, gw_ref, gb_ref, ga_ref, wc_ref, bc_ref,
               lg_ref):
    inv_n = 1.0 / N
    mean = cs_ref[...] * inv_n
    eo2 = cq_ref[...] * inv_n
    a = ga_ref[...]
    var = eo2 - (2.0 * a - a * a) * mean * mean
    inv = lax.rsqrt(var + 1e-5)
    o = out_ref[...]
    sh = o - a * mean
    nrm = gw_ref[...] * sh * inv + gb_ref[...]
    nrm = jnp.maximum(nrm, 0.0)
    lg_ref[...] = jnp.dot(nrm, wc_ref[...], preferred_element_type=f32) + bc_ref[...]


def _epilogue2(out, cs, cq, gw2, gb2, ga2, W_cls_p, b_cls_p):
    r128 = pl.BlockSpec((ROWS, HID), lambda i: (i, 0))
    w_spec = pl.BlockSpec((HID, HID), lambda i: (0, 0))
    one_spec = pl.BlockSpec((1, HID), lambda i: (0, 0))
    return pl.pallas_call(
        _epi2_body,
        grid=(GRID,),
        in_specs=[r128, one_spec, one_spec, one_spec, one_spec, one_spec,
                  w_spec, one_spec],
        out_specs=r128,
        out_shape=jax.ShapeDtypeStruct((N, HID), f32),
    )(out, cs, cq, gw2, gb2, ga2, W_cls_p, b_cls_p)


@jax.jit
def kernel(x, edge_index, edge_attr, W_in, WQ, WK, WV, WE, W_out, b_out,
           gn_weight, gn_bias, gn_alpha, W_cls, b_cls):
    EXP = jnp.kron(jnp.eye(HEADS, dtype=f32), jnp.ones((1, HD), f32))

    h, q, kv0, kv1 = _prework(x, W_in, WQ, WK, WV)
    emb0, emb1 = _prework_edges(edge_attr, WE)
    zeros = jnp.zeros((UPT, HID), f32)
    u0, u1, rs0, rs1 = _edge_phase(zeros, edge_index[0], edge_index[1], q,
                                   emb0, emb1, kv0, kv1)
    u0h = u0.reshape(-1, HALF)[:N]
    u1h = u1.reshape(-1, HALF)[:N]
    rs0u = rs0.reshape(-1, 4)[:N]
    rs1u = rs1.reshape(-1, 4)[:N]
    rsfull = jnp.concatenate([rs0u, rs1u], axis=1)
    out, cs, cq = _epilogue1(u0h, u1h, rsfull, h, W_out, EXP,
                             b_out.reshape(1, HID))
    W_cls_p = jnp.zeros((HID, HID), f32).at[:, :2].set(W_cls)
    b_cls_p = jnp.zeros((1, HID), f32).at[0, :2].set(b_cls)
    logits_p = _epilogue2(out, cs, cq, gn_weight.reshape(1, HID),
                          gn_bias.reshape(1, HID), gn_alpha.reshape(1, HID),
                          W_cls_p, b_cls_p)
    return logits_p[:, :2]
